# Initial kernel scaffold; baseline (speedup 1.0000x reference)
import jax
import jax.numpy as jnp
from jax.experimental import pallas as pl


def kernel(x, edge_index, gcn1_Wrel, gcn1_brel, gcn1_Wroot, gcn2_Wrel, gcn2_brel, gcn2_Wroot, gcn3_Wrel, gcn3_brel, gcn3_Wroot, gcn4_Wrel, gcn4_brel, gcn4_Wroot, fc1_W, fc1_b, fc2_W, fc2_b, fc3_W, fc3_b, fc4_W, fc4_b):
    raise NotImplementedError("write your pallas kernel here")


# speedup vs baseline: 9.6198x; 9.6198x over previous
import functools

import jax
import jax.numpy as jnp
from jax import lax
from jax.experimental import pallas as pl
from jax.experimental.pallas import tpu as pltpu
from jax.experimental.pallas import tpu_sc as plsc

N = 100000
NPAD = 100096
E = 1600000
D = 16
EB = 128
ER = E // EB
NC = 2
NT = 16
NW = NC * NT
RPT = NPAD // NT
BASE_ROWS = ER // NW
EXTRA = ER - BASE_ROWS * NW


def _edge_pass_body(y_hbm, src_hbm, dst_hbm, zeros_hbm, out_hbm,
                    acc, sidx, didx, rows, sem):
    c = lax.axis_index("c")
    s = lax.axis_index("s")
    wid = s * NC + c

    off = pl.multiple_of(s * RPT, 8)
    pltpu.sync_copy(zeros_hbm.at[pl.ds(off, RPT)],
                    acc.at[pl.ds(off, RPT)])
    plsc.subcore_barrier()

    start = wid * BASE_ROWS + jnp.minimum(wid, EXTRA)
    cnt = BASE_ROWS + jnp.where(wid < EXTRA, 1, 0)

    def body(r, carry):
        pltpu.sync_copy(src_hbm.at[r], sidx)
        pltpu.sync_copy(dst_hbm.at[r], didx)
        pltpu.async_copy(y_hbm.at[sidx], rows, sem).wait()
        pltpu.sync_copy(rows, acc.at[didx], add=True)
        return carry

    lax.fori_loop(start, start + cnt, body, 0)
    plsc.subcore_barrier()

    off2 = pl.multiple_of(s * RPT, 8)
    pltpu.sync_copy(acc.at[pl.ds(off2, RPT)],
                    out_hbm.at[c, pl.ds(off2, RPT)])


_edge_pass = functools.partial(
    pl.kernel,
    out_type=jax.ShapeDtypeStruct((NC, NPAD, D), jnp.float32),
    mesh=plsc.VectorSubcoreMesh(core_axis_name="c", subcore_axis_name="s"),
    compiler_params=pltpu.CompilerParams(use_tc_tiling_on_sc=False),
    scratch_types=[
        pltpu.VMEM_SHARED((NPAD, D), jnp.float32),
        pltpu.VMEM((EB,), jnp.int32),
        pltpu.VMEM((EB,), jnp.int32),
        pltpu.VMEM((EB, D), jnp.float32),
        pltpu.SemaphoreType.DMA,
    ],
)(_edge_pass_body)


RB = 4000
GRID = N // RB


def _full(shape):
    return pl.BlockSpec(shape, lambda i: (0,) * len(shape))


def _rows(bs):
    return pl.BlockSpec(bs, lambda i: (i,) + (0,) * (len(bs) - 1))


def _prows():
    return pl.BlockSpec((NC, RB, D), lambda i: (0, i, 0))


def _tc1_body(p_ref, x_ref, wrel_ref, b_ref, wroot_ref, wnext_ref,
              h_ref, y_ref, inv_ref):
    p = p_ref[0] + p_ref[1]
    inv = 1.0 / jnp.maximum(p[:, 1:2], 1.0)
    mean = p[:, 0:1] * inv
    h = jax.nn.softplus(mean @ wrel_ref[...] + b_ref[...]
                        + x_ref[...] @ wroot_ref[...])
    h_ref[...] = h
    y_ref[...] = jnp.pad(h @ wnext_ref[...], ((0, 0), (0, D - 15)))
    inv_ref[...] = inv


def _tc1(p, x, wrel, b, wroot, wnext):
    return pl.pallas_call(
        _tc1_body,
        grid=(GRID,),
        in_specs=[_prows(), _rows((RB, 1)), _full((1, 20)),
                  _full((1, 20)), _full((1, 20)), _full((20, 15))],
        out_specs=[_rows((RB, 20)), _rows((RB, D)), _rows((RB, 1))],
        out_shape=[jax.ShapeDtypeStruct((N, 20), jnp.float32),
                   jax.ShapeDtypeStruct((N, D), jnp.float32),
                   jax.ShapeDtypeStruct((N, 1), jnp.float32)],
    )(p, x, wrel, b, wroot, wnext)


def _tcmid_body(p_ref, h_ref, inv_ref, b_ref, wroot_ref, wnext_ref,
                hout_ref, ynext_ref, *, dmid, dnext):
    mean = (p_ref[0] + p_ref[1])[:, :dmid] * inv_ref[...]
    h = jax.nn.softplus(mean + b_ref[...] + h_ref[...] @ wroot_ref[...])
    hout_ref[...] = h
    ynext_ref[...] = jnp.pad(h @ wnext_ref[...], ((0, 0), (0, D - dnext)))


def _tcmid(p, h, inv, b, wroot, wnext, dprev, dmid, dnext):
    return pl.pallas_call(
        functools.partial(_tcmid_body, dmid=dmid, dnext=dnext),
        grid=(GRID,),
        in_specs=[_prows(), _rows((RB, dprev)), _rows((RB, 1)),
                  _full((1, dmid)), _full((dprev, dmid)),
                  _full((dmid, dnext))],
        out_specs=[_rows((RB, dmid)), _rows((RB, D))],
        out_shape=[jax.ShapeDtypeStruct((N, dmid), jnp.float32),
                   jax.ShapeDtypeStruct((N, D), jnp.float32)],
    )(p, h, inv, b, wroot, wnext)


def _tc4_body(p_ref, h_ref, inv_ref, b_ref, wroot_ref,
              w1_ref, b1_ref, w2_ref, b2_ref, w3_ref, b3_ref, w4_ref, b4_ref,
              out_ref):
    mean = (p_ref[0] + p_ref[1])[:, :10] * inv_ref[...]
    h = jax.nn.softplus(mean + b_ref[...] + h_ref[...] @ wroot_ref[...])
    h = jax.nn.softplus(h @ w1_ref[...] + b1_ref[...])
    h = jax.nn.softplus(h @ w2_ref[...] + b2_ref[...])
    h = jax.nn.softplus(h @ w3_ref[...] + b3_ref[...])
    out_ref[...] = h @ w4_ref[...] + b4_ref[...]


def _tc4(p, h3, inv, b, wroot, w1, b1, w2, b2, w3, b3, w4, b4):
    return pl.pallas_call(
        _tc4_body,
        grid=(GRID,),
        in_specs=[_prows(), _rows((RB, 12)), _rows((RB, 1)),
                  _full((1, 10)), _full((12, 10)),
                  _full((10, 32)), _full((1, 32)),
                  _full((32, 16)), _full((1, 16)),
                  _full((16, 8)), _full((1, 8)),
                  _full((8, 128)), _full((1, 128))],
        out_specs=_rows((RB, 128)),
        out_shape=jax.ShapeDtypeStruct((N, 128), jnp.float32),
    )(p, h3, inv, b, wroot, w1, b1, w2, b2, w3, b3, w4, b4)


def kernel(x, edge_index,
           gcn1_Wrel, gcn1_brel, gcn1_Wroot,
           gcn2_Wrel, gcn2_brel, gcn2_Wroot,
           gcn3_Wrel, gcn3_brel, gcn3_Wroot,
           gcn4_Wrel, gcn4_brel, gcn4_Wroot,
           fc1_W, fc1_b, fc2_W, fc2_b, fc3_W, fc3_b, fc4_W, fc4_b):
    src = edge_index[0].reshape(ER, EB)
    dst = edge_index[1].reshape(ER, EB)
    zeros = jnp.zeros((NPAD, D), jnp.float32)
    y1 = jnp.concatenate(
        [x, jnp.ones((N, 1), jnp.float32), jnp.zeros((N, D - 2), jnp.float32)],
        axis=1)

    p1 = _edge_pass(y1, src, dst, zeros)
    h1, y2, inv = _tc1(p1, x, gcn1_Wrel, gcn1_brel.reshape(1, -1),
                       gcn1_Wroot, gcn2_Wrel)
    p2 = _edge_pass(y2, src, dst, zeros)
    h2, y3 = _tcmid(p2, h1, inv, gcn2_brel.reshape(1, -1), gcn2_Wroot,
                    gcn3_Wrel, 20, 15, 12)
    p3 = _edge_pass(y3, src, dst, zeros)
    h3, y4 = _tcmid(p3, h2, inv, gcn3_brel.reshape(1, -1), gcn3_Wroot,
                    gcn4_Wrel, 15, 12, 10)
    p4 = _edge_pass(y4, src, dst, zeros)
    out = _tc4(p4, h3, inv, gcn4_brel.reshape(1, -1), gcn4_Wroot,
               fc1_W, fc1_b.reshape(1, -1), fc2_W, fc2_b.reshape(1, -1),
               fc3_W, fc3_b.reshape(1, -1), fc4_W, fc4_b.reshape(1, -1))
    return out


# speedup vs baseline: 22.5132x; 2.3403x over previous
import functools

import jax
import jax.numpy as jnp
from jax import lax
from jax.experimental import pallas as pl
from jax.experimental.pallas import tpu as pltpu
from jax.experimental.pallas import tpu_sc as plsc

N = 100000
NPAD = 100096
E = 1600000
D = 16
EB = 128
ER = E // EB
NC = 2
NT = 16
NW = NC * NT
RPT = NPAD // NT
G = 10
NG = ER // G
BASE_G = NG // NW
EXTRA_G = NG - BASE_G * NW


def _edge_pass_body(y_hbm, src_hbm, dst_hbm, zeros_hbm, out_hbm,
                    acc, sidx, didx, rows, sem):
    c = lax.axis_index("c")
    s = lax.axis_index("s")
    wid = s * NC + c

    off = pl.multiple_of(s * RPT, 8)
    pltpu.sync_copy(zeros_hbm.at[pl.ds(off, RPT)],
                    acc.at[pl.ds(off, RPT)])
    plsc.subcore_barrier()

    start = wid * BASE_G + jnp.minimum(wid, EXTRA_G)
    cnt = BASE_G + jnp.where(wid < EXTRA_G, 1, 0)

    def body(g, carry):
        rb = g * G
        pltpu.sync_copy(src_hbm.at[pl.ds(rb, G)], sidx)
        pltpu.sync_copy(dst_hbm.at[pl.ds(rb, G)], didx)
        gd = [pltpu.async_copy(y_hbm.at[sidx.at[j]], rows.at[j], sem)
              for j in range(G)]
        for dsc in gd:
            dsc.wait()
        sd = [pltpu.async_copy(rows.at[j], acc.at[didx.at[j]], sem, add=True)
              for j in range(G)]
        for dsc in sd:
            dsc.wait()
        return carry

    lax.fori_loop(start, start + cnt, body, 0)
    plsc.subcore_barrier()

    off2 = pl.multiple_of(s * RPT, 8)
    pltpu.sync_copy(acc.at[pl.ds(off2, RPT)],
                    out_hbm.at[c, pl.ds(off2, RPT)])


_edge_pass = functools.partial(
    pl.kernel,
    out_type=jax.ShapeDtypeStruct((NC, NPAD, D), jnp.float32),
    mesh=plsc.VectorSubcoreMesh(core_axis_name="c", subcore_axis_name="s"),
    compiler_params=pltpu.CompilerParams(use_tc_tiling_on_sc=False),
    scratch_types=[
        pltpu.VMEM_SHARED((NPAD, D), jnp.float32),
        pltpu.VMEM((G, EB), jnp.int32),
        pltpu.VMEM((G, EB), jnp.int32),
        pltpu.VMEM((G, EB, D), jnp.float32),
        pltpu.SemaphoreType.DMA,
    ],
)(_edge_pass_body)


RB = 4000
GRID = N // RB


def _full(shape):
    return pl.BlockSpec(shape, lambda i: (0,) * len(shape))


def _rows(bs):
    return pl.BlockSpec(bs, lambda i: (i,) + (0,) * (len(bs) - 1))


def _prows():
    return pl.BlockSpec((NC, RB, D), lambda i: (0, i, 0))


def _tc1_body(p_ref, x_ref, wrel_ref, b_ref, wroot_ref, wnext_ref,
              h_ref, y_ref, inv_ref):
    p = p_ref[0] + p_ref[1]
    inv = 1.0 / jnp.maximum(p[:, 1:2], 1.0)
    mean = p[:, 0:1] * inv
    h = jax.nn.softplus(mean @ wrel_ref[...] + b_ref[...]
                        + x_ref[...] @ wroot_ref[...])
    h_ref[...] = h
    y_ref[...] = jnp.pad(h @ wnext_ref[...], ((0, 0), (0, D - 15)))
    inv_ref[...] = inv


def _tc1(p, x, wrel, b, wroot, wnext):
    return pl.pallas_call(
        _tc1_body,
        grid=(GRID,),
        in_specs=[_prows(), _rows((RB, 1)), _full((1, 20)),
                  _full((1, 20)), _full((1, 20)), _full((20, 15))],
        out_specs=[_rows((RB, 20)), _rows((RB, D)), _rows((RB, 1))],
        out_shape=[jax.ShapeDtypeStruct((N, 20), jnp.float32),
                   jax.ShapeDtypeStruct((N, D), jnp.float32),
                   jax.ShapeDtypeStruct((N, 1), jnp.float32)],
    )(p, x, wrel, b, wroot, wnext)


def _tcmid_body(p_ref, h_ref, inv_ref, b_ref, wroot_ref, wnext_ref,
                hout_ref, ynext_ref, *, dmid, dnext):
    mean = (p_ref[0] + p_ref[1])[:, :dmid] * inv_ref[...]
    h = jax.nn.softplus(mean + b_ref[...] + h_ref[...] @ wroot_ref[...])
    hout_ref[...] = h
    ynext_ref[...] = jnp.pad(h @ wnext_ref[...], ((0, 0), (0, D - dnext)))


def _tcmid(p, h, inv, b, wroot, wnext, dprev, dmid, dnext):
    return pl.pallas_call(
        functools.partial(_tcmid_body, dmid=dmid, dnext=dnext),
        grid=(GRID,),
        in_specs=[_prows(), _rows((RB, dprev)), _rows((RB, 1)),
                  _full((1, dmid)), _full((dprev, dmid)),
                  _full((dmid, dnext))],
        out_specs=[_rows((RB, dmid)), _rows((RB, D))],
        out_shape=[jax.ShapeDtypeStruct((N, dmid), jnp.float32),
                   jax.ShapeDtypeStruct((N, D), jnp.float32)],
    )(p, h, inv, b, wroot, wnext)


def _tc4_body(p_ref, h_ref, inv_ref, b_ref, wroot_ref,
              w1_ref, b1_ref, w2_ref, b2_ref, w3_ref, b3_ref, w4_ref, b4_ref,
              out_ref):
    mean = (p_ref[0] + p_ref[1])[:, :10] * inv_ref[...]
    h = jax.nn.softplus(mean + b_ref[...] + h_ref[...] @ wroot_ref[...])
    h = jax.nn.softplus(h @ w1_ref[...] + b1_ref[...])
    h = jax.nn.softplus(h @ w2_ref[...] + b2_ref[...])
    h = jax.nn.softplus(h @ w3_ref[...] + b3_ref[...])
    out_ref[...] = h @ w4_ref[...] + b4_ref[...]


def _tc4(p, h3, inv, b, wroot, w1, b1, w2, b2, w3, b3, w4, b4):
    return pl.pallas_call(
        _tc4_body,
        grid=(GRID,),
        in_specs=[_prows(), _rows((RB, 12)), _rows((RB, 1)),
                  _full((1, 10)), _full((12, 10)),
                  _full((10, 32)), _full((1, 32)),
                  _full((32, 16)), _full((1, 16)),
                  _full((16, 8)), _full((1, 8)),
                  _full((8, 128)), _full((1, 128))],
        out_specs=_rows((RB, 128)),
        out_shape=jax.ShapeDtypeStruct((N, 128), jnp.float32),
    )(p, h3, inv, b, wroot, w1, b1, w2, b2, w3, b3, w4, b4)


def kernel(x, edge_index,
           gcn1_Wrel, gcn1_brel, gcn1_Wroot,
           gcn2_Wrel, gcn2_brel, gcn2_Wroot,
           gcn3_Wrel, gcn3_brel, gcn3_Wroot,
           gcn4_Wrel, gcn4_brel, gcn4_Wroot,
           fc1_W, fc1_b, fc2_W, fc2_b, fc3_W, fc3_b, fc4_W, fc4_b):
    src = edge_index[0].reshape(ER, EB)
    dst = edge_index[1].reshape(ER, EB)
    zeros = jnp.zeros((NPAD, D), jnp.float32)
    y1 = jnp.concatenate(
        [x, jnp.ones((N, 1), jnp.float32), jnp.zeros((N, D - 2), jnp.float32)],
        axis=1)

    p1 = _edge_pass(y1, src, dst, zeros)
    h1, y2, inv = _tc1(p1, x, gcn1_Wrel, gcn1_brel.reshape(1, -1),
                       gcn1_Wroot, gcn2_Wrel)
    p2 = _edge_pass(y2, src, dst, zeros)
    h2, y3 = _tcmid(p2, h1, inv, gcn2_brel.reshape(1, -1), gcn2_Wroot,
                    gcn3_Wrel, 20, 15, 12)
    p3 = _edge_pass(y3, src, dst, zeros)
    h3, y4 = _tcmid(p3, h2, inv, gcn3_brel.reshape(1, -1), gcn3_Wroot,
                    gcn4_Wrel, 15, 12, 10)
    p4 = _edge_pass(y4, src, dst, zeros)
    out = _tc4(p4, h3, inv, gcn4_brel.reshape(1, -1), gcn4_Wroot,
               fc1_W, fc1_b.reshape(1, -1), fc2_W, fc2_b.reshape(1, -1),
               fc3_W, fc3_b.reshape(1, -1), fc4_W, fc4_b.reshape(1, -1))
    return out


# speedup vs baseline: 36.9556x; 1.6415x over previous
import functools

import jax
import jax.numpy as jnp
from jax import lax
from jax.experimental import pallas as pl
from jax.experimental.pallas import tpu as pltpu
from jax.experimental.pallas import tpu_sc as plsc

N = 100000
NPAD = 100096
E = 1600000
D = 16
EB = 128
ER = E // EB
NC = 2
NT = 16
NW = NC * NT
RPT = NPAD // NT
G = 10
NG = ER // G
BASE_G = NG // NW
EXTRA_G = NG - BASE_G * NW


def _edge_pass_body(y_hbm, src_hbm, dst_hbm, zeros_hbm, out_hbm,
                    acc, sidx, didx, rows, sem):
    c = lax.axis_index("c")
    s = lax.axis_index("s")
    wid = s * NC + c

    off = pl.multiple_of(s * RPT, 8)
    pltpu.sync_copy(zeros_hbm.at[pl.ds(off, RPT)],
                    acc.at[pl.ds(off, RPT)])
    plsc.subcore_barrier()

    start = wid * BASE_G + jnp.minimum(wid, EXTRA_G)
    cnt = BASE_G + jnp.where(wid < EXTRA_G, 1, 0)

    def body(g, carry):
        rb = g * G
        pltpu.sync_copy(src_hbm.at[pl.ds(rb, G)], sidx)
        pltpu.sync_copy(dst_hbm.at[pl.ds(rb, G)], didx)
        gd = [pltpu.async_copy(y_hbm.at[sidx.at[j]], rows.at[j], sem)
              for j in range(G)]
        for dsc in gd:
            dsc.wait()
        sd = [pltpu.async_copy(rows.at[j], acc.at[didx.at[j]], sem, add=True)
              for j in range(G)]
        for dsc in sd:
            dsc.wait()
        return carry

    lax.fori_loop(start, start + cnt, body, 0)
    plsc.subcore_barrier()

    off2 = pl.multiple_of(s * RPT, 8)
    pltpu.sync_copy(acc.at[pl.ds(off2, RPT)],
                    out_hbm.at[c, pl.ds(off2, RPT)])


_edge_pass = functools.partial(
    pl.kernel,
    out_type=jax.ShapeDtypeStruct((NC, NPAD, D), jnp.float32),
    mesh=plsc.VectorSubcoreMesh(core_axis_name="c", subcore_axis_name="s"),
    compiler_params=pltpu.CompilerParams(use_tc_tiling_on_sc=False),
    scratch_types=[
        pltpu.VMEM_SHARED((NPAD, D), jnp.float32),
        pltpu.VMEM((G, EB), jnp.int32),
        pltpu.VMEM((G, EB), jnp.int32),
        pltpu.VMEM((G, EB, D), jnp.float32),
        pltpu.SemaphoreType.DMA,
    ],
)(_edge_pass_body)


PR = NPAD // 8
PB = 736
UR = PB * 8
GRID = PR // PB


def _full(shape):
    return pl.BlockSpec(shape, lambda i: (0,) * len(shape))


def _rowsP():
    return pl.BlockSpec((PB, 128), lambda i: (i, 0))


def _prowsP():
    return pl.BlockSpec((NC, PB, 128), lambda i: (0, i, 0))


def _tc1_body(p_ref, y1p_ref, bsum_ref, bcnt_ref, vrelA_ref, vbA_ref,
              vrootA_ref, vrelB_ref, vbB_ref, vrootB_ref,
              w2relA_ref, w2relB_ref, w2rootA_ref, w2rootB_ref,
              y2p_ref, r2p_ref, invp_ref):
    p8 = p_ref[0] + p_ref[1]
    inv = 1.0 / jnp.maximum(p8 @ bcnt_ref[...], 1.0)
    mean_b = (p8 @ bsum_ref[...]) * inv
    x_b = y1p_ref[...] @ bsum_ref[...]
    h1a = jax.nn.softplus(mean_b * vrelA_ref[...] + vbA_ref[...]
                          + x_b * vrootA_ref[...])
    h1b = jax.nn.softplus(mean_b * vrelB_ref[...] + vbB_ref[...]
                          + x_b * vrootB_ref[...])
    y2p_ref[...] = h1a @ w2relA_ref[...] + h1b @ w2relB_ref[...]
    r2p_ref[...] = h1a @ w2rootA_ref[...] + h1b @ w2rootB_ref[...]
    invp_ref[...] = inv


def _tc1(p, y1p, consts):
    return pl.pallas_call(
        _tc1_body,
        grid=(GRID,),
        in_specs=[_prowsP(), _rowsP()] + [_full(c.shape) for c in consts],
        out_specs=[_rowsP(), _rowsP(), _rowsP()],
        out_shape=[jax.ShapeDtypeStruct((PR, 128), jnp.float32)] * 3,
    )(p, y1p, *consts)


def _tcmid_body(p_ref, rp_ref, invp_ref, bb_ref, bwrel_ref, bwroot_ref,
                yp_ref, rnp_ref):
    pre = (p_ref[0] + p_ref[1]) * invp_ref[...] + rp_ref[...] + bb_ref[...]
    h8 = jax.nn.softplus(pre)
    yp_ref[...] = h8 @ bwrel_ref[...]
    rnp_ref[...] = h8 @ bwroot_ref[...]


def _tcmid(p, rp, invp, bb, bwrel, bwroot):
    return pl.pallas_call(
        _tcmid_body,
        grid=(GRID,),
        in_specs=[_prowsP(), _rowsP(), _rowsP(), _full((1, 128)),
                  _full((128, 128)), _full((128, 128))],
        out_specs=[_rowsP(), _rowsP()],
        out_shape=[jax.ShapeDtypeStruct((PR, 128), jnp.float32)] * 2,
    )(p, rp, invp, bb, bwrel, bwroot)


def _tc4_body(p_ref, rp_ref, invp_ref, bb_ref,
              bw1_ref, bb1_ref, bw2_ref, bb2_ref, bw3_ref, bb3_ref,
              bw4_ref, bb4_ref, out_ref):
    pre = (p_ref[0] + p_ref[1]) * invp_ref[...] + rp_ref[...] + bb_ref[...]
    h = jax.nn.softplus(pre)
    h = jax.nn.softplus(h @ bw1_ref[...] + bb1_ref[...])
    h = jax.nn.softplus(h @ bw2_ref[...] + bb2_ref[...])
    h = jax.nn.softplus(h @ bw3_ref[...] + bb3_ref[...])
    o = h @ bw4_ref[...] + bb4_ref[...]
    out_ref[...] = o.reshape(UR, 128)


def _tc4(p, rp, invp, bb, bw1, bb1, bw2, bb2, bw3, bb3, bw4, bb4):
    return pl.pallas_call(
        _tc4_body,
        grid=(GRID,),
        in_specs=[_prowsP(), _rowsP(), _rowsP(), _full((1, 128)),
                  _full((128, 256)), _full((1, 256)),
                  _full((256, 128)), _full((1, 128)),
                  _full((128, 64)), _full((1, 64)),
                  _full((64, 1024)), _full((1, 1024))],
        out_specs=pl.BlockSpec((UR, 128), lambda i: (i, 0)),
        out_shape=jax.ShapeDtypeStruct((N, 128), jnp.float32),
    )(p, rp, invp, bb, bw1, bb1, bw2, bb2, bw3, bb3, bw4, bb4)


def _padw(w, rows, cols):
    return jnp.pad(w, ((0, rows - w.shape[0]), (0, cols - w.shape[1])))


def _bd8(w, rows, cols):
    return jnp.kron(jnp.eye(8, dtype=jnp.float32), _padw(w, rows, cols))


def _tileb(b, width):
    return jnp.tile(jnp.pad(b, (0, width - b.shape[0])), 8).reshape(1, -1)


def kernel(x, edge_index,
           gcn1_Wrel, gcn1_brel, gcn1_Wroot,
           gcn2_Wrel, gcn2_brel, gcn2_Wroot,
           gcn3_Wrel, gcn3_brel, gcn3_Wroot,
           gcn4_Wrel, gcn4_brel, gcn4_Wroot,
           fc1_W, fc1_b, fc2_W, fc2_b, fc3_W, fc3_b, fc4_W, fc4_b):
    src = edge_index[0].reshape(ER, EB)
    dst = edge_index[1].reshape(ER, EB)
    zeros = jnp.zeros((NPAD, D), jnp.float32)
    y1 = jnp.concatenate(
        [x, jnp.ones((N, 1), jnp.float32), jnp.zeros((N, D - 2), jnp.float32)],
        axis=1)
    y1 = jnp.pad(y1, ((0, NPAD - N), (0, 0)))

    e0 = jnp.zeros((D, D), jnp.float32).at[0, :].set(1.0)
    e1 = jnp.zeros((D, D), jnp.float32).at[1, :].set(1.0)
    bsum = jnp.kron(jnp.eye(8, dtype=jnp.float32), e0)
    bcnt = jnp.kron(jnp.eye(8, dtype=jnp.float32), e1)
    w1rel, w1root, b1 = gcn1_Wrel[0], gcn1_Wroot[0], gcn1_brel
    vrelA = _tileb(w1rel[:16], D)
    vbA = _tileb(b1[:16], D)
    vrootA = _tileb(w1root[:16], D)
    vrelB = _tileb(w1rel[16:], D)
    vbB = _tileb(b1[16:], D)
    vrootB = _tileb(w1root[16:], D)
    w2relA = _bd8(gcn2_Wrel[:16], D, D)
    w2relB = _bd8(gcn2_Wrel[16:], D, D)
    w2rootA = _bd8(gcn2_Wroot[:16], D, D)
    w2rootB = _bd8(gcn2_Wroot[16:], D, D)
    tc1_consts = [bsum, bcnt, vrelA, vbA, vrootA, vrelB, vbB, vrootB,
                  w2relA, w2relB, w2rootA, w2rootB]

    p1 = _edge_pass(y1, src, dst, zeros)
    y2p, r2p, invp = _tc1(p1.reshape(NC, PR, 128), y1.reshape(PR, 128),
                          tc1_consts)
    p2 = _edge_pass(y2p.reshape(NPAD, D), src, dst, zeros)
    y3p, r3p = _tcmid(p2.reshape(NC, PR, 128), r2p, invp,
                      _tileb(gcn2_brel, D),
                      _bd8(gcn3_Wrel, D, D), _bd8(gcn3_Wroot, D, D))
    p3 = _edge_pass(y3p.reshape(NPAD, D), src, dst, zeros)
    y4p, r4p = _tcmid(p3.reshape(NC, PR, 128), r3p, invp,
                      _tileb(gcn3_brel, D),
                      _bd8(gcn4_Wrel, D, D), _bd8(gcn4_Wroot, D, D))
    p4 = _edge_pass(y4p.reshape(NPAD, D), src, dst, zeros)
    out = _tc4(p4.reshape(NC, PR, 128), r4p, invp, _tileb(gcn4_brel, D),
               _bd8(fc1_W, D, 32), _tileb(fc1_b, 32),
               _bd8(fc2_W, 32, D), _tileb(fc2_b, D),
               _bd8(fc3_W, D, 8), _tileb(fc3_b, 8),
               _bd8(fc4_W, 8, 128), _tileb(fc4_b, 128))
    return out


# speedup vs baseline: 52.1361x; 1.4108x over previous
import functools

import jax
import jax.numpy as jnp
from jax import lax
from jax.experimental import pallas as pl
from jax.experimental.pallas import tpu as pltpu
from jax.experimental.pallas import tpu_sc as plsc

N = 100000
NPAD = 100096
E = 1600000
D = 16
EB = 128
ER = E // EB
NC = 2
NT = 16
NW = NC * NT
RPT = NPAD // NT
G = 5
NG = ER // G
BASE_G = NG // NW
EXTRA_G = NG - BASE_G * NW


def _edge_pass_body(y_hbm, edge_hbm, zeros_hbm, out_hbm,
                    acc, eidxA, eidxB, rowsA, rowsB,
                    semGA, semGB, semSA, semSB):
    c = lax.axis_index("c")
    s = lax.axis_index("s")
    wid = s * NC + c

    off = pl.multiple_of(s * RPT, 8)
    pltpu.sync_copy(zeros_hbm.at[pl.ds(off, RPT)],
                    acc.at[pl.ds(off, RPT)])
    plsc.subcore_barrier()

    start = wid * BASE_G + jnp.minimum(wid, EXTRA_G)
    cnt = BASE_G + jnp.where(wid < EXTRA_G, 1, 0)

    def fire_gathers(g, eidx, rows, semG):
        pltpu.sync_copy(edge_hbm.at[:, pl.ds(g * G, G)], eidx)
        for j in range(G):
            pltpu.async_copy(y_hbm.at[eidx.at[0, j]], rows.at[j], semG)

    def wait_gathers(eidx, rows, semG):
        for j in range(G):
            pltpu.make_async_copy(y_hbm.at[eidx.at[0, j]], rows.at[j],
                                  semG).wait()

    def fire_scatters(eidx, rows, semS):
        for j in range(G):
            pltpu.async_copy(rows.at[j], acc.at[eidx.at[1, j]], semS,
                             add=True)

    def wait_scatters(eidx, rows, semS):
        for j in range(G):
            pltpu.make_async_copy(rows.at[j], acc.at[eidx.at[1, j]],
                                  semS).wait()

    def steady(t, g, eC, rC, gC, sC, eN, rN, gN, sN):
        @pl.when(t > 0)
        def _():
            wait_scatters(eN, rN, sN)

        @pl.when(t + 1 < cnt)
        def _():
            fire_gathers(g + 1, eN, rN, gN)

        wait_gathers(eC, rC, gC)
        fire_scatters(eC, rC, sC)

    fire_gathers(start, eidxA, rowsA, semGA)

    def body(t, carry):
        g = start + t

        @pl.when(t % 2 == 0)
        def _():
            steady(t, g, eidxA, rowsA, semGA, semSA,
                   eidxB, rowsB, semGB, semSB)

        @pl.when(t % 2 == 1)
        def _():
            steady(t, g, eidxB, rowsB, semGB, semSB,
                   eidxA, rowsA, semGA, semSA)

        return carry

    lax.fori_loop(0, cnt, body, 0)

    @pl.when(cnt % 2 == 1)
    def _():
        wait_scatters(eidxA, rowsA, semSA)

    @pl.when(cnt % 2 == 0)
    def _():
        wait_scatters(eidxB, rowsB, semSB)

    plsc.subcore_barrier()

    off2 = pl.multiple_of(s * RPT, 8)
    pltpu.sync_copy(acc.at[pl.ds(off2, RPT)],
                    out_hbm.at[c, pl.ds(off2, RPT)])


_edge_pass = functools.partial(
    pl.kernel,
    out_type=jax.ShapeDtypeStruct((NC, NPAD, D), jnp.float32),
    mesh=plsc.VectorSubcoreMesh(core_axis_name="c", subcore_axis_name="s"),
    compiler_params=pltpu.CompilerParams(use_tc_tiling_on_sc=False),
    scratch_types=[
        pltpu.VMEM_SHARED((NPAD, D), jnp.float32),
        pltpu.VMEM((2, G, EB), jnp.int32),
        pltpu.VMEM((2, G, EB), jnp.int32),
        pltpu.VMEM((G, EB, D), jnp.float32),
        pltpu.VMEM((G, EB, D), jnp.float32),
        pltpu.SemaphoreType.DMA,
        pltpu.SemaphoreType.DMA,
        pltpu.SemaphoreType.DMA,
        pltpu.SemaphoreType.DMA,
    ],
)(_edge_pass_body)


PR = NPAD // 8
PB = 736
UR = PB * 8
GRID = PR // PB


def _full(shape):
    return pl.BlockSpec(shape, lambda i: (0,) * len(shape))


def _rowsP():
    return pl.BlockSpec((PB, 128), lambda i: (i, 0))


def _prowsP():
    return pl.BlockSpec((NC, PB, 128), lambda i: (0, i, 0))


def _tc1_body(p_ref, y1p_ref, bsum_ref, bcnt_ref, vrelA_ref, vbA_ref,
              vrootA_ref, vrelB_ref, vbB_ref, vrootB_ref,
              w2relA_ref, w2relB_ref, w2rootA_ref, w2rootB_ref,
              y2p_ref, r2p_ref, invp_ref):
    p8 = p_ref[0] + p_ref[1]
    inv = 1.0 / jnp.maximum(p8 @ bcnt_ref[...], 1.0)
    mean_b = (p8 @ bsum_ref[...]) * inv
    x_b = y1p_ref[...] @ bsum_ref[...]
    h1a = jax.nn.softplus(mean_b * vrelA_ref[...] + vbA_ref[...]
                          + x_b * vrootA_ref[...])
    h1b = jax.nn.softplus(mean_b * vrelB_ref[...] + vbB_ref[...]
                          + x_b * vrootB_ref[...])
    y2p_ref[...] = h1a @ w2relA_ref[...] + h1b @ w2relB_ref[...]
    r2p_ref[...] = h1a @ w2rootA_ref[...] + h1b @ w2rootB_ref[...]
    invp_ref[...] = inv


def _tc1(p, y1p, consts):
    return pl.pallas_call(
        _tc1_body,
        grid=(GRID,),
        in_specs=[_prowsP(), _rowsP()] + [_full(c.shape) for c in consts],
        out_specs=[_rowsP(), _rowsP(), _rowsP()],
        out_shape=[jax.ShapeDtypeStruct((PR, 128), jnp.float32)] * 3,
    )(p, y1p, *consts)


def _tcmid_body(p_ref, rp_ref, invp_ref, bb_ref, bwrel_ref, bwroot_ref,
                yp_ref, rnp_ref):
    pre = (p_ref[0] + p_ref[1]) * invp_ref[...] + rp_ref[...] + bb_ref[...]
    h8 = jax.nn.softplus(pre)
    yp_ref[...] = h8 @ bwrel_ref[...]
    rnp_ref[...] = h8 @ bwroot_ref[...]


def _tcmid(p, rp, invp, bb, bwrel, bwroot):
    return pl.pallas_call(
        _tcmid_body,
        grid=(GRID,),
        in_specs=[_prowsP(), _rowsP(), _rowsP(), _full((1, 128)),
                  _full((128, 128)), _full((128, 128))],
        out_specs=[_rowsP(), _rowsP()],
        out_shape=[jax.ShapeDtypeStruct((PR, 128), jnp.float32)] * 2,
    )(p, rp, invp, bb, bwrel, bwroot)


def _tc4_body(p_ref, rp_ref, invp_ref, bb_ref,
              bw1_ref, bb1_ref, bw2_ref, bb2_ref, bw3_ref, bb3_ref,
              bw4_ref, bb4_ref, out_ref):
    pre = (p_ref[0] + p_ref[1]) * invp_ref[...] + rp_ref[...] + bb_ref[...]
    h = jax.nn.softplus(pre)
    h = jax.nn.softplus(h @ bw1_ref[...] + bb1_ref[...])
    h = jax.nn.softplus(h @ bw2_ref[...] + bb2_ref[...])
    h = jax.nn.softplus(h @ bw3_ref[...] + bb3_ref[...])
    o = h @ bw4_ref[...] + bb4_ref[...]
    out_ref[...] = o.reshape(UR, 128)


def _tc4(p, rp, invp, bb, bw1, bb1, bw2, bb2, bw3, bb3, bw4, bb4):
    return pl.pallas_call(
        _tc4_body,
        grid=(GRID,),
        in_specs=[_prowsP(), _rowsP(), _rowsP(), _full((1, 128)),
                  _full((128, 256)), _full((1, 256)),
                  _full((256, 128)), _full((1, 128)),
                  _full((128, 64)), _full((1, 64)),
                  _full((64, 1024)), _full((1, 1024))],
        out_specs=pl.BlockSpec((UR, 128), lambda i: (i, 0)),
        out_shape=jax.ShapeDtypeStruct((N, 128), jnp.float32),
    )(p, rp, invp, bb, bw1, bb1, bw2, bb2, bw3, bb3, bw4, bb4)


def _padw(w, rows, cols):
    return jnp.pad(w, ((0, rows - w.shape[0]), (0, cols - w.shape[1])))


def _bd8(w, rows, cols):
    return jnp.kron(jnp.eye(8, dtype=jnp.float32), _padw(w, rows, cols))


def _tileb(b, width):
    return jnp.tile(jnp.pad(b, (0, width - b.shape[0])), 8).reshape(1, -1)


def kernel(x, edge_index,
           gcn1_Wrel, gcn1_brel, gcn1_Wroot,
           gcn2_Wrel, gcn2_brel, gcn2_Wroot,
           gcn3_Wrel, gcn3_brel, gcn3_Wroot,
           gcn4_Wrel, gcn4_brel, gcn4_Wroot,
           fc1_W, fc1_b, fc2_W, fc2_b, fc3_W, fc3_b, fc4_W, fc4_b):
    edges = edge_index.reshape(2, ER, EB)
    zeros = jnp.zeros((NPAD, D), jnp.float32)
    y1 = jnp.concatenate(
        [x, jnp.ones((N, 1), jnp.float32), jnp.zeros((N, D - 2), jnp.float32)],
        axis=1)
    y1 = jnp.pad(y1, ((0, NPAD - N), (0, 0)))

    e0 = jnp.zeros((D, D), jnp.float32).at[0, :].set(1.0)
    e1 = jnp.zeros((D, D), jnp.float32).at[1, :].set(1.0)
    bsum = jnp.kron(jnp.eye(8, dtype=jnp.float32), e0)
    bcnt = jnp.kron(jnp.eye(8, dtype=jnp.float32), e1)
    w1rel, w1root, b1 = gcn1_Wrel[0], gcn1_Wroot[0], gcn1_brel
    vrelA = _tileb(w1rel[:16], D)
    vbA = _tileb(b1[:16], D)
    vrootA = _tileb(w1root[:16], D)
    vrelB = _tileb(w1rel[16:], D)
    vbB = _tileb(b1[16:], D)
    vrootB = _tileb(w1root[16:], D)
    w2relA = _bd8(gcn2_Wrel[:16], D, D)
    w2relB = _bd8(gcn2_Wrel[16:], D, D)
    w2rootA = _bd8(gcn2_Wroot[:16], D, D)
    w2rootB = _bd8(gcn2_Wroot[16:], D, D)
    tc1_consts = [bsum, bcnt, vrelA, vbA, vrootA, vrelB, vbB, vrootB,
                  w2relA, w2relB, w2rootA, w2rootB]

    p1 = _edge_pass(y1, edges, zeros)
    y2p, r2p, invp = _tc1(p1.reshape(NC, PR, 128), y1.reshape(PR, 128),
                          tc1_consts)
    p2 = _edge_pass(y2p.reshape(NPAD, D), edges, zeros)
    y3p, r3p = _tcmid(p2.reshape(NC, PR, 128), r2p, invp,
                      _tileb(gcn2_brel, D),
                      _bd8(gcn3_Wrel, D, D), _bd8(gcn3_Wroot, D, D))
    p3 = _edge_pass(y3p.reshape(NPAD, D), edges, zeros)
    y4p, r4p = _tcmid(p3.reshape(NC, PR, 128), r3p, invp,
                      _tileb(gcn3_brel, D),
                      _bd8(gcn4_Wrel, D, D), _bd8(gcn4_Wroot, D, D))
    p4 = _edge_pass(y4p.reshape(NPAD, D), edges, zeros)
    out = _tc4(p4.reshape(NC, PR, 128), r4p, invp, _tileb(gcn4_brel, D),
               _bd8(fc1_W, D, 32), _tileb(fc1_b, 32),
               _bd8(fc2_W, 32, D), _tileb(fc2_b, D),
               _bd8(fc3_W, D, 8), _tileb(fc3_b, 8),
               _bd8(fc4_W, 8, 128), _tileb(fc4_b, 128))
    return out
